# baseline (speedup 1.0000x reference)
import functools

import jax
import jax.numpy as jnp
from jax import lax
from jax.experimental import pallas as pl
from jax.experimental.pallas import tpu as pltpu
from jax.experimental.pallas import tpu_sc as plsc

NC = 2
NS = 16
L = 16
NW = NC * NS
K = 128
IG = 8


def _sc_spmm(h, colw, roww, valw):
    N, D = h.shape
    cpw = colw.shape[1]
    ngroups = cpw // IG
    mesh = plsc.VectorSubcoreMesh(core_axis_name="c", subcore_axis_name="s")
    rps = ((-(-N // NS)) + 7) // 8 * 8

    @functools.partial(
        pl.kernel,
        out_type=jax.ShapeDtypeStruct((NC, N, D), jnp.float32),
        mesh=mesh,
        scratch_types=[
            pltpu.VMEM((2 * IG, K), jnp.int32),
            pltpu.VMEM((2 * IG, K), jnp.int32),
            pltpu.VMEM((2 * IG * K,), jnp.float32),
            pltpu.VMEM((2, K, D), jnp.float32),
            pltpu.VMEM_SHARED((N, D), jnp.float32),
            pltpu.SemaphoreType.DMA,
            pltpu.SemaphoreType.DMA,
            pltpu.SemaphoreType.DMA,
        ],
    )
    def k(h_hbm, col_hbm, row_hbm, val_hbm, out_hbm,
          col_v, dst_v, val_v, rows_v, acc, gsem0, gsem1, isem):
        cid = lax.axis_index("c")
        sid = lax.axis_index("s")
        wid = sid * NC + cid
        gsems = (gsem0, gsem1)

        def zero_body(i, carry):
            for d in range(D // L):
                rows_v[0, i, pl.ds(d * L, L)] = jnp.zeros((L,), jnp.float32)
            return carry

        lax.fori_loop(0, K, zero_body, 0)
        base = jnp.minimum(sid * rps, N - rps)
        off = 0
        while off < rps:
            sz = min(K, rps - off)
            pltpu.sync_copy(rows_v.at[0, pl.ds(0, sz)],
                            acc.at[pl.ds(base + off, sz)])
            off += sz
        plsc.subcore_barrier()

        pltpu.sync_copy(col_hbm.at[wid, pl.ds(0, IG)], col_v.at[pl.ds(0, IG)])
        pltpu.sync_copy(row_hbm.at[wid, pl.ds(0, IG)], dst_v.at[pl.ds(0, IG)])
        pltpu.sync_copy(val_hbm.at[wid, pl.ds(0, IG * K)],
                        val_v.at[pl.ds(0, IG * K)])

        def idx_loads(gq1, nbase):
            return (
                (col_hbm.at[wid, pl.ds(gq1 * IG, IG)],
                 col_v.at[pl.ds(nbase, IG)]),
                (row_hbm.at[wid, pl.ds(gq1 * IG, IG)],
                 dst_v.at[pl.ds(nbase, IG)]),
                (val_hbm.at[wid, pl.ds(gq1 * IG * K, IG * K)],
                 val_v.at[pl.ds(nbase * K, IG * K)]),
            )

        def group_body(gq, carry):
            par = lax.rem(gq, 2)
            rbase = par * IG
            nbase = (1 - par) * IG

            @pl.when(gq + 1 < ngroups)
            def _():
                for src, dst in idx_loads(gq + 1, nbase):
                    pltpu.async_copy(src, dst, isem)

            for b in range(IG):
                ch = gq * IG + b
                bb = 0
                buf = rows_v.at[bb]
                pltpu.async_copy(h_hbm.at[col_v.at[rbase + b]], buf,
                                 gsems[bb]).wait()

                vbase = (rbase + b) * K

                def scale_body(e16, c2, bb=bb, vbase=vbase):
                    vblock = val_v[pl.ds(vbase + e16 * L, L)]
                    for j in range(L):
                        vv = jnp.full((L,), vblock[j])
                        e = e16 * L + j
                        for d in range(D // L):
                            sl = pl.ds(d * L, L)
                            rows_v[bb, e, sl] = rows_v[bb, e, sl] * vv
                    return c2

                lax.fori_loop(0, K // L, scale_body, 0)

                pltpu.sync_copy(buf, acc.at[dst_v.at[rbase + b]], add=True)

                if b == IG - 3:
                    @pl.when(gq + 1 < ngroups)
                    def _():
                        for src, dst in idx_loads(gq + 1, nbase):
                            pltpu.make_async_copy(src, dst, isem).wait()

            return carry

        lax.fori_loop(0, ngroups, group_body, 0)
        plsc.subcore_barrier()

        pltpu.sync_copy(acc.at[pl.ds(base, rps)],
                        out_hbm.at[cid, pl.ds(base, rps)])

    return k(h, colw, roww, valw)


def _add_partials(p):
    _, N, D = p.shape

    def body(a_ref, b_ref, o_ref):
        o_ref[...] = a_ref[...] + b_ref[...]

    bn = N
    for cand in (2000, 1000, 500, 250, 128, 8):
        if N % cand == 0:
            bn = cand
            break
    grid = N // bn
    spec = pl.BlockSpec((bn, D), lambda i: (i, 0))
    return pl.pallas_call(
        body,
        out_shape=jax.ShapeDtypeStruct((N, D), jnp.float32),
        grid=(grid,),
        in_specs=[spec, spec],
        out_specs=spec,
    )(p[0], p[1])


def kernel(x, edge_index, edge_vals):
    N, D = x.shape
    E = edge_vals.shape[0]
    row = edge_index[0].astype(jnp.int32)
    col = edge_index[1].astype(jnp.int32)
    vals = edge_vals.astype(jnp.float32)

    cpw = -(-E // (NW * K))
    cpw = ((cpw + IG - 1) // IG) * IG
    epad = NW * K * cpw
    pad = epad - E
    if pad:
        row = jnp.concatenate([row, jnp.zeros((pad,), jnp.int32)])
        col = jnp.concatenate([col, jnp.zeros((pad,), jnp.int32)])
        vals = jnp.concatenate([vals, jnp.zeros((pad,), jnp.float32)])
    roww = row.reshape(NW, cpw, K)
    colw = col.reshape(NW, cpw, K)
    valw = vals.reshape(NW, cpw * K)

    out = x
    for _ in range(2):
        out = _add_partials(_sc_spmm(out, colw, roww, valw))
    return out


# speedup vs baseline: 1.2690x; 1.2690x over previous
import functools

import jax
import jax.numpy as jnp
from jax import lax
from jax.experimental import pallas as pl
from jax.experimental.pallas import tpu as pltpu
from jax.experimental.pallas import tpu_sc as plsc

NC = 2
NS = 16
L = 16
NW = NC * NS
K = 80
IG = 8
NB = 4


def _sc_spmm(h, colw, roww, valw):
    N, D = h.shape
    cpw = colw.shape[1]
    ngroups = cpw // IG
    mesh = plsc.VectorSubcoreMesh(core_axis_name="c", subcore_axis_name="s")
    rps = ((-(-N // NS)) + 7) // 8 * 8
    ring = 2 * IG

    @functools.partial(
        pl.kernel,
        out_type=jax.ShapeDtypeStruct((NC, N, D), jnp.float32),
        mesh=mesh,
        scratch_types=[
            pltpu.VMEM((ring, K), jnp.int32),
            pltpu.VMEM((ring, K), jnp.int32),
            pltpu.VMEM((ring * K,), jnp.float32),
            pltpu.VMEM((NB, K, D), jnp.float32),
            pltpu.VMEM_SHARED((N, D), jnp.float32),
            [pltpu.SemaphoreType.DMA] * NB,
            [pltpu.SemaphoreType.DMA] * NB,
            pltpu.SemaphoreType.DMA,
        ],
    )
    def k(h_hbm, col_hbm, row_hbm, val_hbm, out_hbm,
          col_v, dst_v, val_v, rows_v, acc, gsems, ssems, isem):
        cid = lax.axis_index("c")
        sid = lax.axis_index("s")
        wid = sid * NC + cid

        def zero_body(i, carry):
            for d in range(D // L):
                rows_v[0, i, pl.ds(d * L, L)] = jnp.zeros((L,), jnp.float32)
            return carry

        lax.fori_loop(0, K, zero_body, 0)
        base = jnp.minimum(sid * rps, N - rps)
        off = 0
        while off < rps:
            sz = min(K, rps - off)
            pltpu.sync_copy(rows_v.at[0, pl.ds(0, sz)],
                            acc.at[pl.ds(base + off, sz)])
            off += sz
        plsc.subcore_barrier()

        def idx_loads(g1):
            half = lax.rem(g1, 2) * IG
            return (
                (col_hbm.at[wid, pl.ds(g1 * IG, IG)],
                 col_v.at[pl.ds(half, IG)]),
                (row_hbm.at[wid, pl.ds(g1 * IG, IG)],
                 dst_v.at[pl.ds(half, IG)]),
                (val_hbm.at[wid, pl.ds(g1 * IG * K, IG * K)],
                 val_v.at[pl.ds(half * K, IG * K)]),
            )

        def gather(ch, bb):
            rr = lax.rem(ch, ring)
            return (h_hbm.at[col_v.at[rr]], rows_v.at[bb], gsems[bb])

        def scatter(ch, bb):
            rr = lax.rem(ch, ring)
            return (rows_v.at[bb], acc.at[dst_v.at[rr]], ssems[bb])

        for src, dst in idx_loads(0):
            pltpu.sync_copy(src, dst)
        pltpu.async_copy(*gather(0, 0))
        pltpu.async_copy(*gather(1, 1))

        def quad_body(q, carry):
            qpar = lax.rem(q, 2)
            for b in range(NB):
                ch = q * NB + b

                if b == 0:
                    @pl.when((qpar == 0) & (ch + IG < cpw))
                    def _():
                        for src, dst in idx_loads(ch // IG + 1):
                            pltpu.async_copy(src, dst, isem)

                if b == 2:
                    @pl.when((qpar == 1) & (ch + 2 < cpw))
                    def _():
                        for src, dst in idx_loads(ch // IG + 1):
                            pltpu.make_async_copy(src, dst, isem).wait()

                nb = (b + 2) % NB

                @pl.when(ch >= 2)
                def _(nb=nb):
                    pltpu.make_async_copy(*scatter(ch - 2, nb)).wait()

                @pl.when(ch + 2 < cpw)
                def _(nb=nb):
                    pltpu.async_copy(*gather(ch + 2, nb))

                pltpu.make_async_copy(*gather(ch, b)).wait()

                rr = lax.rem(ch, ring)

                def scale_body(e16, c2, b=b, rr=rr):
                    vblock = val_v[pl.ds(rr * K + e16 * L, L)]
                    for j in range(L):
                        vv = jnp.full((L,), vblock[j])
                        e = e16 * L + j
                        for d in range(D // L):
                            sl = pl.ds(d * L, L)
                            rows_v[b, e, sl] = rows_v[b, e, sl] * vv
                    return c2

                lax.fori_loop(0, K // L, scale_body, 0)
                pltpu.async_copy(*scatter(ch, b), add=True)
            return carry

        lax.fori_loop(0, cpw // NB, quad_body, 0)

        for ch in (cpw - 2, cpw - 1):
            pltpu.make_async_copy(*scatter(ch, ch % NB)).wait()
        plsc.subcore_barrier()
        pltpu.sync_copy(acc.at[pl.ds(base, rps)],
                        out_hbm.at[cid, pl.ds(base, rps)])

    return k(h, colw, roww, valw)


def _add_partials(p):
    _, N, D = p.shape

    def body(a_ref, b_ref, o_ref):
        o_ref[...] = a_ref[...] + b_ref[...]

    bn = N
    for cand in (2000, 1000, 500, 250, 128, 8):
        if N % cand == 0:
            bn = cand
            break
    grid = N // bn
    spec = pl.BlockSpec((bn, D), lambda i: (i, 0))
    return pl.pallas_call(
        body,
        out_shape=jax.ShapeDtypeStruct((N, D), jnp.float32),
        grid=(grid,),
        in_specs=[spec, spec],
        out_specs=spec,
    )(p[0], p[1])


def kernel(x, edge_index, edge_vals):
    N, D = x.shape
    E = edge_vals.shape[0]
    row = edge_index[0].astype(jnp.int32)
    col = edge_index[1].astype(jnp.int32)
    vals = edge_vals.astype(jnp.float32)

    cpw = -(-E // (NW * K))
    cpw = ((cpw + IG - 1) // IG) * IG
    epad = NW * K * cpw
    pad = epad - E
    if pad:
        row = jnp.concatenate([row, jnp.zeros((pad,), jnp.int32)])
        col = jnp.concatenate([col, jnp.zeros((pad,), jnp.int32)])
        vals = jnp.concatenate([vals, jnp.zeros((pad,), jnp.float32)])
    roww = row.reshape(NW, cpw, K)
    colw = col.reshape(NW, cpw, K)
    valw = vals.reshape(NW, cpw * K)

    out = x
    for _ in range(2):
        out = _add_partials(_sc_spmm(out, colw, roww, valw))
    return out
